# baseline (speedup 1.0000x reference)
import functools

import jax
import jax.numpy as jnp
from jax import lax
from jax.experimental import pallas as pl
from jax.experimental.pallas import tpu as pltpu
from jax.experimental.pallas import tpu_sc as plsc

NC = 2
NS = 16
NW = NC * NS
LANES = 16
CH = 64
NSLOT = 5
EPS = 1e-5


def _lane_sum(v):
    dnums = lax.GatherDimensionNumbers(
        offset_dims=(), collapsed_slice_dims=(0,), start_index_map=(0,))
    for k in (1, 2, 4, 8):
        perm = (lax.iota(jnp.int32, LANES) ^ k).reshape(LANES, 1)
        v = v + lax.gather(v, perm, dnums, (1,),
                           mode=lax.GatherScatterMode.PROMISE_IN_BOUNDS)
    return v


def _rsqrt(x):
    i = lax.bitcast_convert_type(x, jnp.int32)
    i = jnp.int32(0x5F3759DF) - lax.shift_right_logical(i, 1)
    y = lax.bitcast_convert_type(i, jnp.float32)
    for _ in range(3):
        y = y * (1.5 - 0.5 * x * y * y)
    return y


def _make_sc_kernel(n_rows, seq_len, hidden, n_pt):
    rows_pw = n_rows // NW
    nch = rows_pw // CH
    nvec = hidden // LANES
    nsegw = rows_pw // 32
    nsegw = -(-nsegw // LANES) * LANES
    mesh = plsc.VectorSubcoreMesh(
        core_axis_name="c", subcore_axis_name="s",
        num_cores=NC, num_subcores=NS)

    @functools.partial(
        pl.kernel,
        out_type=jax.ShapeDtypeStruct((n_rows, hidden), jnp.float32),
        mesh=mesh,
        scratch_types=[
            pltpu.VMEM((nch, CH), jnp.int32),
            pltpu.VMEM((n_pt, hidden), jnp.float32),
            pltpu.VMEM((NSLOT, CH, hidden), jnp.float32),
            pltpu.VMEM((2, hidden), jnp.float32),
            pltpu.VMEM((nsegw // LANES, LANES), jnp.int32),
            pltpu.SMEM((nsegw,), jnp.int32),
            pltpu.SemaphoreType.DMA,
            pltpu.SemaphoreType.DMA,
        ],
    )
    def sc_kernel(tok_hbm, segw_hbm, word_hbm, pt_hbm, gb_hbm, out_hbm,
                  tok_v, pt_v, buf, gb_v, segw_v, segw_s, sem_w, sem_o):
        wid = lax.axis_index("s") * NC + lax.axis_index("c")
        pltpu.sync_copy(tok_hbm.at[wid], tok_v)
        pltpu.sync_copy(segw_hbm.at[wid], segw_v)
        pltpu.sync_copy(gb_hbm, gb_v)
        pltpu.sync_copy(pt_hbm, pt_v)
        for k in range(nsegw // LANES):
            vw = segw_v[k]
            for u in range(LANES):
                segw_s[LANES * k + u] = vw[u]

        g = [gb_v[0, pl.ds(LANES * j, LANES)] for j in range(nvec)]
        bta = [gb_v[1, pl.ds(LANES * j, LANES)] for j in range(nvec)]

        def issue_gather(c, slot):
            pltpu.async_copy(word_hbm.at[tok_v.at[c]], buf.at[slot], sem_w)

        def wait_gather(c, slot):
            pltpu.make_async_copy(word_hbm.at[tok_v.at[c]],
                                  buf.at[slot], sem_w).wait()

        def wait_scatter(slot):
            pltpu.make_async_copy(buf.at[slot],
                                  out_hbm.at[pl.ds(0, CH)], sem_o).wait()

        for c0 in range(3):
            issue_gather(c0, c0)

        def row_norm(slot, c, pbase, i):
            r = c * CH + i
            seg = (segw_s[lax.shift_right_logical(r, 5)]
                   >> lax.bitwise_and(r, 31)) & 1
            p = pbase + i
            prow = jnp.where(p >= seq_len, p - seq_len, p)
            ptrow = seg * seq_len + prow
            x = [buf[slot, i, pl.ds(LANES * j, LANES)]
                 + pt_v[ptrow, pl.ds(LANES * j, LANES)]
                 for j in range(nvec)]
            s = x[0]
            sq = x[0] * x[0]
            for j in range(1, nvec):
                s = s + x[j]
                sq = sq + x[j] * x[j]
            mean = _lane_sum(s) * (1.0 / hidden)
            ex2 = _lane_sum(sq) * (1.0 / hidden)
            var = ex2 - mean * mean
            rstd = _rsqrt(var + EPS)
            rg = [rstd * gj for gj in g]
            for j in range(nvec):
                buf[slot, i, pl.ds(LANES * j, LANES)] = (
                    x[j] * rg[j] + (bta[j] - mean * rg[j]))

        def chunk_body(c, slot):
            @pl.when(jnp.logical_and(c >= 2, c + 3 < nch))
            def _():
                wait_scatter((c + 3) % NSLOT)

            @pl.when(c + 3 < nch)
            def _():
                issue_gather(c + 3, (c + 3) % NSLOT)

            wait_gather(c, slot)
            pbase = lax.rem(c * CH, seq_len)

            def rows(ii, _):
                row_norm(slot, c, pbase, 2 * ii)
                row_norm(slot, c, pbase, 2 * ii + 1)
                return 0
            lax.fori_loop(0, CH // 2, rows, 0)
            pltpu.async_copy(buf.at[slot],
                             out_hbm.at[pl.ds(wid * rows_pw + c * CH, CH)],
                             sem_o)

        def outer(gi, _):
            for u in range(NSLOT):
                chunk_body(NSLOT * gi + u, u)
            return 0
        lax.fori_loop(0, nch // NSLOT, outer, 0)

        for _ in range(NSLOT):
            wait_scatter(0)

    return sc_kernel


def kernel(tokens, segments, word_emb, pos_emb, type_emb, ln_gamma, ln_beta):
    bsz, seq_len = tokens.shape
    vocab, hidden = word_emb.shape
    n_rows = bsz * seq_len
    pt = (type_emb[:, None, :] + pos_emb[None, :seq_len, :]).reshape(-1, hidden)
    tok = tokens.reshape(NW, -1, CH).astype(jnp.int32)
    segb = segments.reshape(NW, -1, 32).astype(jnp.int32)
    shifts = jnp.arange(32, dtype=jnp.int32)
    segw = jnp.sum(segb << shifts[None, None, :], axis=-1, dtype=jnp.int32)
    npad = -(-segw.shape[1] // 16) * 16 - segw.shape[1]
    segw = jnp.pad(segw, ((0, 0), (0, npad))).reshape(NW, -1, 16)
    gb = jnp.stack([ln_gamma, ln_beta]).astype(jnp.float32)
    fn = _make_sc_kernel(n_rows, seq_len, hidden, pt.shape[0])
    out = fn(tok, segw, word_emb.astype(jnp.float32), pt, gb)
    return out.reshape(bsz, seq_len, hidden)


# speedup vs baseline: 1.5759x; 1.5759x over previous
import functools

import jax
import jax.numpy as jnp
from jax import lax
from jax.experimental import pallas as pl
from jax.experimental.pallas import tpu as pltpu
from jax.experimental.pallas import tpu_sc as plsc

NC = 2
NS = 16
NW = NC * NS
LANES = 16
CH = 128
EPS = 1e-5

_DNUMS = lax.GatherDimensionNumbers(
    offset_dims=(), collapsed_slice_dims=(0,), start_index_map=(0,))


def _permute(v, perm):
    return lax.gather(v, perm.reshape(LANES, 1), _DNUMS, (1,),
                      mode=lax.GatherScatterMode.PROMISE_IN_BOUNDS)


def _lane_sum(v):
    for k in (1, 2, 4, 8):
        v = v + _permute(v, lax.iota(jnp.int32, LANES) ^ k)
    return v


def _rsqrt(x):
    i = lax.bitcast_convert_type(x, jnp.int32)
    i = jnp.int32(0x5F3759DF) - lax.shift_right_logical(i, 1)
    y = lax.bitcast_convert_type(i, jnp.float32)
    for _ in range(3):
        y = y * (1.5 - 0.5 * x * y * y)
    return y


def _make_sc_kernel(n_rows, seq_len, hidden, n_pt):
    rows_pw = n_rows // NW
    nch = rows_pw // CH
    nvec = hidden // LANES
    mesh = plsc.VectorSubcoreMesh(
        core_axis_name="c", subcore_axis_name="s",
        num_cores=NC, num_subcores=NS)

    @functools.partial(
        pl.kernel,
        out_type=jax.ShapeDtypeStruct((n_rows, hidden), jnp.float32),
        mesh=mesh,
        scratch_types=[
            pltpu.VMEM((nch, CH), jnp.int32),
            pltpu.VMEM((nch, CH), jnp.int32),
            pltpu.VMEM((2, CH), jnp.int32),
            pltpu.VMEM((2, CH, hidden), jnp.float32),
            pltpu.VMEM((2, CH, hidden), jnp.float32),
            pltpu.VMEM((2, CH, hidden), jnp.float32),
            pltpu.VMEM((2, hidden), jnp.float32),
            pltpu.SemaphoreType.DMA,
            pltpu.SemaphoreType.DMA,
            pltpu.SemaphoreType.DMA,
        ],
    )
    def sc_kernel(tok_hbm, seg_hbm, word_hbm, pt_hbm, gb_hbm, out_hbm,
                  tok_v, seg_v, ptidx_v, wbuf, ptbuf, obuf, gb_v,
                  sem_w, sem_p, sem_o):
        wid = lax.axis_index("s") * NC + lax.axis_index("c")
        pltpu.sync_copy(tok_hbm.at[wid], tok_v)
        pltpu.sync_copy(seg_hbm.at[wid], seg_v)
        pltpu.sync_copy(gb_hbm, gb_v)

        g = [gb_v[0, pl.ds(LANES * j, LANES)] for j in range(nvec)]
        bta = [gb_v[1, pl.ds(LANES * j, LANES)] for j in range(nvec)]

        def fill_ptidx(c, slot):
            pbase = lax.rem(c * CH, seq_len)
            for k in range(CH // LANES):
                seg16 = seg_v[c, pl.ds(LANES * k, LANES)]
                p16 = pbase + LANES * k + lax.iota(jnp.int32, LANES)
                prow16 = jnp.where(p16 >= seq_len, p16 - seq_len, p16)
                ptidx_v[slot, pl.ds(LANES * k, LANES)] = (
                    seg16 * seq_len + prow16)

        def issue_gathers(c, slot):
            pltpu.async_copy(word_hbm.at[tok_v.at[c]], wbuf.at[slot], sem_w)
            pltpu.async_copy(pt_hbm.at[ptidx_v.at[slot]], ptbuf.at[slot],
                             sem_p)

        for c0 in range(2):
            fill_ptidx(c0, c0)
            issue_gathers(c0, c0)

        def row_norm(slot, i):
            x = [wbuf[slot, i, pl.ds(LANES * j, LANES)]
                 + ptbuf[slot, i, pl.ds(LANES * j, LANES)]
                 for j in range(nvec)]
            s = x[0]
            sq = x[0] * x[0]
            for j in range(1, nvec):
                s = s + x[j]
                sq = sq + x[j] * x[j]
            mean = _lane_sum(s) * (1.0 / hidden)
            ex2 = _lane_sum(sq) * (1.0 / hidden)
            var = ex2 - mean * mean
            rstd = _rsqrt(var + EPS)
            c0 = -mean * rstd
            for j in range(nvec):
                obuf[slot, i, pl.ds(LANES * j, LANES)] = (
                    (x[j] * rstd + c0) * g[j] + bta[j])

        def outer(gi, _):
            for slot in range(2):
                c = 2 * gi + slot
                pltpu.make_async_copy(word_hbm.at[tok_v.at[c]],
                                      wbuf.at[slot], sem_w).wait()
                pltpu.make_async_copy(pt_hbm.at[ptidx_v.at[slot]],
                                      ptbuf.at[slot], sem_p).wait()

                @pl.when(c >= 2)
                def _():
                    pltpu.make_async_copy(obuf.at[slot],
                                          out_hbm.at[pl.ds(0, CH)],
                                          sem_o).wait()

                def rows(i, _):
                    row_norm(slot, 2 * i)
                    row_norm(slot, 2 * i + 1)
                    return 0
                lax.fori_loop(0, CH // 2, rows, 0)

                row0 = wid * rows_pw + c * CH
                pltpu.async_copy(obuf.at[slot],
                                 out_hbm.at[pl.ds(row0, CH)], sem_o)

                @pl.when(c + 2 < nch)
                def _():
                    fill_ptidx(c + 2, slot)
                    issue_gathers(c + 2, slot)
            return 0

        lax.fori_loop(0, nch // 2, outer, 0)

        for _ in range(2):
            pltpu.make_async_copy(obuf.at[0], out_hbm.at[pl.ds(0, CH)],
                                  sem_o).wait()

    return sc_kernel


def kernel(tokens, segments, word_emb, pos_emb, type_emb, ln_gamma, ln_beta):
    bsz, seq_len = tokens.shape
    vocab, hidden = word_emb.shape
    n_rows = bsz * seq_len
    pt = (type_emb[:, None, :] + pos_emb[None, :seq_len, :]).reshape(-1, hidden)
    tok = tokens.reshape(NW, -1, CH).astype(jnp.int32)
    seg = segments.reshape(NW, -1, CH).astype(jnp.int32)
    gb = jnp.stack([ln_gamma, ln_beta]).astype(jnp.float32)
    fn = _make_sc_kernel(n_rows, seq_len, hidden, pt.shape[0])
    out = fn(tok, seg, word_emb.astype(jnp.float32), pt, gb)
    return out.reshape(bsz, seq_len, hidden)
